# baseline (speedup 1.0000x reference)
import functools

import jax
import jax.numpy as jnp
from jax.experimental import pallas as pl
from jax.experimental.pallas import tpu as pltpu

_BB = 4


def _asmk_kernel(x_ref, c_ref, w_ref, out_ref):
    bb, n, d = x_ref.shape
    k = c_ref.shape[0]
    x = x_ref[...].reshape(bb * n, d)
    c = c_ref[...]

    x2 = jnp.sum(x * x, axis=1, keepdims=True)
    c2 = jnp.sum(c * c, axis=1)[None, :]
    xcn = jax.lax.dot_general(
        x * -2.0, c, (((1,), (1,)), ((), ())),
        preferred_element_type=jnp.float32)
    d2 = (x2 + c2) + xcn

    nearest = jnp.argmin(d2, axis=1)
    min_d = jnp.sqrt(jnp.maximum(jnp.min(d2, axis=1), 0.0))

    md = min_d.reshape(bb, n)
    mean = jnp.mean(md, axis=1, keepdims=True)
    std = jnp.sqrt(jnp.sum((md - mean) ** 2, axis=1, keepdims=True) / (n - 1))
    thr = mean + std
    mask = (md < thr).astype(jnp.float32).reshape(bb * n)

    kiota = jax.lax.broadcasted_iota(jnp.int32, (bb * n, k), 1)
    onehot = (nearest[:, None] == kiota)
    contrib = jnp.where(onehot, mask[:, None], 0.0)
    hist = jnp.sum(contrib.reshape(bb, n, k), axis=1)

    asmk = w_ref[...] * hist
    norm = jnp.sqrt(jnp.sum(asmk * asmk, axis=1, keepdims=True))
    out_ref[...] = (asmk / jnp.maximum(norm, 1e-12)).reshape(bb, 1, k)


@functools.partial(jax.jit, static_argnames=())
def kernel(x, centroids, weights):
    B, N, D = x.shape
    K = centroids.shape[0]
    w2d = weights.reshape(1, K)
    return pl.pallas_call(
        _asmk_kernel,
        grid=(B // _BB,),
        in_specs=[
            pl.BlockSpec((_BB, N, D), lambda b: (b, 0, 0)),
            pl.BlockSpec((K, D), lambda b: (0, 0)),
            pl.BlockSpec((1, K), lambda b: (0, 0)),
        ],
        out_specs=pl.BlockSpec((_BB, 1, K), lambda b: (b, 0, 0)),
        out_shape=jax.ShapeDtypeStruct((B, 1, K), x.dtype),
        compiler_params=pltpu.CompilerParams(
            dimension_semantics=("parallel",)),
    )(x, centroids, w2d).reshape(B, K)


# speedup vs baseline: 1.4035x; 1.4035x over previous
import functools

import jax
import jax.numpy as jnp
from jax.experimental import pallas as pl
from jax.experimental.pallas import tpu as pltpu

_BB = 2


def _asmk_kernel(x_ref, c_ref, w_ref, out_ref):
    bb, n, d = x_ref.shape
    k = c_ref.shape[0]
    x = x_ref[...].reshape(bb * n, d)
    c = c_ref[...]

    x2 = jnp.sum(x * x, axis=1, keepdims=True)
    c2 = jnp.sum(c * c, axis=1, keepdims=True)
    onesx = jnp.ones_like(x2)
    onesc = jnp.ones_like(c2)
    xa = jnp.concatenate([x * -2.0, x2, onesx], axis=1)
    ca = jnp.concatenate([c, onesc, c2], axis=1)
    d2 = jax.lax.dot_general(
        xa, ca, (((1,), (1,)), ((), ())),
        preferred_element_type=jnp.float32)

    min2 = jnp.min(d2, axis=1)
    min_d = jnp.sqrt(jnp.maximum(min2, 0.0))

    md = min_d.reshape(bb, n)
    mean = jnp.mean(md, axis=1, keepdims=True)
    std = jnp.sqrt(jnp.sum((md - mean) ** 2, axis=1, keepdims=True) / (n - 1))
    mask = (md < mean + std).astype(jnp.float32).reshape(bb * n)

    onehot = jnp.where(d2 == min2[:, None], 1.0, 0.0)
    eqcount = jnp.sum(onehot, axis=1)
    contrib = onehot * (mask / eqcount)[:, None]
    hist = jnp.sum(contrib.reshape(bb, n, k), axis=1)

    asmk = w_ref[...] * hist
    norm = jnp.sqrt(jnp.sum(asmk * asmk, axis=1, keepdims=True))
    out_ref[...] = (asmk / jnp.maximum(norm, 1e-12)).reshape(bb, 1, k)


@functools.partial(jax.jit, static_argnames=())
def kernel(x, centroids, weights):
    B, N, D = x.shape
    K = centroids.shape[0]
    w2d = weights.reshape(1, K)
    return pl.pallas_call(
        _asmk_kernel,
        grid=(B // _BB,),
        in_specs=[
            pl.BlockSpec((_BB, N, D), lambda b: (b, 0, 0)),
            pl.BlockSpec((K, D), lambda b: (0, 0)),
            pl.BlockSpec((1, K), lambda b: (0, 0)),
        ],
        out_specs=pl.BlockSpec((_BB, 1, K), lambda b: (b, 0, 0)),
        out_shape=jax.ShapeDtypeStruct((B, 1, K), x.dtype),
        compiler_params=pltpu.CompilerParams(
            dimension_semantics=("parallel",)),
    )(x, centroids, w2d).reshape(B, K)
